# Initial kernel scaffold; baseline (speedup 1.0000x reference)
import jax
import jax.numpy as jnp
from jax.experimental import pallas as pl


def kernel(x, emb, W, b):
    raise NotImplementedError("write your pallas kernel here")


# speedup vs baseline: 1.5478x; 1.5478x over previous
import functools

import jax
import jax.numpy as jnp
from jax import lax
from jax.experimental import pallas as pl
from jax.experimental.pallas import tpu as pltpu
from jax.experimental.pallas import tpu_sc as plsc

B = 4096
L = 50
E = 64
EP = 128
H = 512
N = B * L

NC, NS = 2, 16
NW = NC * NS
ROWS_PER_W = N // NW
CHUNK = 128
N_CHUNKS = ROWS_PER_W // CHUNK

BM = 256


def _sc_gather(emb_pad, idx):
    mesh = plsc.VectorSubcoreMesh(core_axis_name="c", subcore_axis_name="s")

    @functools.partial(
        pl.kernel,
        mesh=mesh,
        out_type=jax.ShapeDtypeStruct((N, EP), jnp.float32),
        scratch_types=[
            pltpu.VMEM((ROWS_PER_W,), jnp.int32),
            pltpu.VMEM((CHUNK, EP), jnp.float32),
            pltpu.SemaphoreType.DMA,
        ],
    )
    def k(emb_hbm, idx_hbm, out_hbm, idx_v, rows_v, sem):
        wid = lax.axis_index("s") * NC + lax.axis_index("c")
        base = wid * ROWS_PER_W
        pltpu.sync_copy(idx_hbm.at[pl.ds(base, ROWS_PER_W)], idx_v)

        @pl.loop(0, N_CHUNKS)
        def _(j):
            off = j * CHUNK
            pltpu.async_copy(
                emb_hbm.at[idx_v.at[pl.ds(off, CHUNK)]], rows_v, sem
            ).wait()
            pltpu.sync_copy(rows_v, out_hbm.at[pl.ds(base + off, CHUNK)])

    return k(emb_pad, idx)


def _mm_body(x_ref, w_ref, b_ref, o_ref):
    li = pl.program_id(1)
    part = jnp.dot(
        x_ref[...].astype(jnp.bfloat16),
        w_ref[...],
        preferred_element_type=jnp.float32,
    )

    @pl.when(li == 0)
    def _():
        o_ref[...] = part

    @pl.when(li != 0)
    def _():
        o_ref[...] += part

    @pl.when(li == L - 1)
    def _():
        o_ref[...] = jnp.maximum(o_ref[...] + b_ref[...], 0.0)


def _tc_matmul(g, Wp, b):
    nbt = B // BM
    return pl.pallas_call(
        _mm_body,
        grid=(nbt, L),
        in_specs=[
            pl.BlockSpec((BM, EP), lambda i, l: (l * nbt + i, 0)),
            pl.BlockSpec((EP, H), lambda i, l: (l, 0)),
            pl.BlockSpec((1, H), lambda i, l: (0, 0)),
        ],
        out_specs=pl.BlockSpec((BM, H), lambda i, l: (i, 0)),
        out_shape=jax.ShapeDtypeStruct((B, H), jnp.float32),
    )(g, Wp, b.reshape(1, H))


def kernel(x, emb, W, b):
    idx = x.astype(jnp.int32).T.reshape(-1)
    emb_pad = jnp.pad(emb, ((0, 0), (0, EP - E)))
    Wp = jnp.concatenate(
        [
            W.reshape(L, E, H).astype(jnp.bfloat16),
            jnp.zeros((L, EP - E, H), jnp.bfloat16),
        ],
        axis=1,
    ).reshape(L * EP, H)
    g = _sc_gather(emb_pad, idx)
    return _tc_matmul(g, Wp, b)


# speedup vs baseline: 4.2443x; 2.7422x over previous
import functools

import jax
import jax.numpy as jnp
from jax import lax
from jax.experimental import pallas as pl
from jax.experimental.pallas import tpu as pltpu
from jax.experimental.pallas import tpu_sc as plsc

B = 4096
L = 50
E = 64
EP = 128
H = 512
N = B * L

P = 10
LP = L // P
NR = N // P
PW = P * EP

NC, NS = 2, 16
NW = NC * NS
ROWS_PER_W = NR // NW
CHUNK = 64
N_CHUNKS = ROWS_PER_W // CHUNK

BM = 256


def _sc_gather(emb_pad, idx_r):
    mesh = plsc.VectorSubcoreMesh(core_axis_name="c", subcore_axis_name="s")

    @functools.partial(
        pl.kernel,
        mesh=mesh,
        out_type=jax.ShapeDtypeStruct((NR, PW), jnp.float32),
        scratch_types=[
            pltpu.VMEM((P, ROWS_PER_W), jnp.int32),
            pltpu.VMEM((CHUNK, PW), jnp.float32),
            pltpu.SemaphoreType.DMA,
        ],
    )
    def k(emb_hbm, idx_hbm, out_hbm, idx_v, buf_v, sem):
        wid = lax.axis_index("s") * NC + lax.axis_index("c")
        base = wid * ROWS_PER_W
        pltpu.sync_copy(idx_hbm.at[:, pl.ds(base, ROWS_PER_W)], idx_v)

        @pl.loop(0, N_CHUNKS)
        def _(j):
            off = j * CHUNK
            copies = [
                pltpu.async_copy(
                    emb_hbm.at[idx_v.at[p, pl.ds(off, CHUNK)]],
                    buf_v.at[:, pl.ds(p * EP, EP)],
                    sem,
                )
                for p in range(P)
            ]
            for c in copies:
                c.wait()
            pltpu.sync_copy(buf_v, out_hbm.at[pl.ds(base + off, CHUNK)])

    return k(emb_pad, idx_r)


def _mm_body(x_ref, w_ref, b_ref, o_ref):
    li = pl.program_id(1)
    part = jnp.dot(
        x_ref[...].astype(jnp.bfloat16),
        w_ref[...],
        preferred_element_type=jnp.float32,
    )

    @pl.when(li == 0)
    def _():
        o_ref[...] = part

    @pl.when(li != 0)
    def _():
        o_ref[...] += part

    @pl.when(li == LP - 1)
    def _():
        o_ref[...] = jnp.maximum(o_ref[...] + b_ref[...], 0.0)


def _tc_matmul(g, Wp, b):
    nbt = B // BM
    return pl.pallas_call(
        _mm_body,
        grid=(nbt, LP),
        in_specs=[
            pl.BlockSpec((BM, PW), lambda i, l: (l * nbt + i, 0)),
            pl.BlockSpec((PW, H), lambda i, l: (l, 0)),
            pl.BlockSpec((1, H), lambda i, l: (0, 0)),
        ],
        out_specs=pl.BlockSpec((BM, H), lambda i, l: (i, 0)),
        out_shape=jax.ShapeDtypeStruct((B, H), jnp.float32),
    )(g, Wp, b.reshape(1, H))


def kernel(x, emb, W, b):
    idx_r = (
        x.astype(jnp.int32).T.reshape(LP, P, B).transpose(1, 0, 2).reshape(P, NR)
    )
    emb_pad = jnp.pad(emb, ((0, 0), (0, EP - E)))
    Wp = jnp.concatenate(
        [
            W.reshape(L, E, H).astype(jnp.bfloat16),
            jnp.zeros((L, EP - E, H), jnp.bfloat16),
        ],
        axis=1,
    ).reshape(L * EP, H)
    g = _sc_gather(emb_pad, idx_r)
    return _tc_matmul(g, Wp, b)


# speedup vs baseline: 5.4141x; 1.2756x over previous
import functools

import jax
import jax.numpy as jnp
from jax import lax
from jax.experimental import pallas as pl
from jax.experimental.pallas import tpu as pltpu
from jax.experimental.pallas import tpu_sc as plsc

B = 4096
L = 50
E = 64
EP = 128
H = 512
N = B * L

P = 10
LP = L // P
PW = P * EP
K = L * EP

NC, NS = 2, 16
NW = NC * NS
B_PER_W = B // NW
CHUNK = 64
N_CHUNKS = B_PER_W // CHUNK

BM = 256


def _sc_gather(emb_pad, idx_r):
    mesh = plsc.VectorSubcoreMesh(core_axis_name="c", subcore_axis_name="s")

    @functools.partial(
        pl.kernel,
        mesh=mesh,
        out_type=jax.ShapeDtypeStruct((B, K), jnp.float32),
        scratch_types=[
            pltpu.VMEM((P, LP * B_PER_W), jnp.int32),
            pltpu.VMEM((CHUNK, PW), jnp.float32),
            pltpu.SemaphoreType.DMA,
        ],
    )
    def k(emb_hbm, idx_hbm, out_hbm, idx_v, buf_v, sem):
        wid = lax.axis_index("s") * NC + lax.axis_index("c")
        b0 = wid * B_PER_W
        @pl.loop(0, LP)
        def _(l):
            pltpu.sync_copy(
                idx_hbm.at[:, pl.ds(l * B + b0, B_PER_W)],
                idx_v.at[:, pl.ds(l * B_PER_W, B_PER_W)],
            )

        @pl.loop(0, LP)
        def _(l):
            @pl.loop(0, N_CHUNKS)
            def _(j):
                off = j * CHUNK
                copies = [
                    pltpu.async_copy(
                        emb_hbm.at[idx_v.at[p, pl.ds(l * B_PER_W + off, CHUNK)]],
                        buf_v.at[:, pl.ds(p * EP, EP)],
                        sem,
                    )
                    for p in range(P)
                ]
                for c in copies:
                    c.wait()
                pltpu.sync_copy(
                    buf_v,
                    out_hbm.at[pl.ds(b0 + off, CHUNK), pl.ds(l * PW, PW)],
                )

    return k(emb_pad, idx_r)


def _mm_body(x_ref, w_ref, b_ref, o_ref):
    acc = jnp.dot(
        x_ref[...].astype(jnp.bfloat16),
        w_ref[...],
        preferred_element_type=jnp.float32,
    )
    o_ref[...] = jnp.maximum(acc + b_ref[...], 0.0)


def _tc_matmul(g, Wp, b):
    return pl.pallas_call(
        _mm_body,
        grid=(B // BM,),
        in_specs=[
            pl.BlockSpec((BM, K), lambda i: (i, 0)),
            pl.BlockSpec((K, H), lambda i: (0, 0)),
            pl.BlockSpec((1, H), lambda i: (0, 0)),
        ],
        out_specs=pl.BlockSpec((BM, H), lambda i: (i, 0)),
        out_shape=jax.ShapeDtypeStruct((B, H), jnp.float32),
    )(g, Wp, b.reshape(1, H))


def kernel(x, emb, W, b):
    idx_r = (
        x.astype(jnp.int32).T.reshape(LP, P, B).transpose(1, 0, 2).reshape(P, LP * B)
    )
    emb_pad = jnp.pad(emb, ((0, 0), (0, EP - E)))
    Wp = jnp.concatenate(
        [
            W.reshape(L, E, H).astype(jnp.bfloat16),
            jnp.zeros((L, EP - E, H), jnp.bfloat16),
        ],
        axis=1,
    ).reshape(K, H)
    g = _sc_gather(emb_pad, idx_r)
    return _tc_matmul(g, Wp, b)


# speedup vs baseline: 5.5514x; 1.0254x over previous
import functools

import jax
import jax.numpy as jnp
from jax import lax
from jax.experimental import pallas as pl
from jax.experimental.pallas import tpu as pltpu
from jax.experimental.pallas import tpu_sc as plsc

B = 4096
L = 50
E = 64
EP = 128
H = 512

P = 10
LP = L // P
PW = P * EP
K = L * EP

NC, NS = 2, 16
NW = NC * NS
B_PER_W = B // NW
CHUNK = 32
ROUNDS = (B_PER_W // CHUNK) * LP
IDX_PER_W = L * B_PER_W

BM = 256


def _sc_gather(emb_pad, idx_r):
    mesh = plsc.VectorSubcoreMesh(core_axis_name="c", subcore_axis_name="s")

    @functools.partial(
        pl.kernel,
        mesh=mesh,
        out_type=jax.ShapeDtypeStruct((B, K), jnp.float32),
        scratch_types=[
            pltpu.VMEM((IDX_PER_W,), jnp.int32),
            pltpu.VMEM((CHUNK, PW), jnp.float32),
            pltpu.VMEM((CHUNK, PW), jnp.float32),
            pltpu.SemaphoreType.DMA,
            pltpu.SemaphoreType.DMA,
            pltpu.SemaphoreType.DMA,
            pltpu.SemaphoreType.DMA,
        ],
    )
    def k(emb_hbm, idx_hbm, out_hbm, idx_v, buf0, buf1, g0, g1, w0, w1):
        wid = lax.axis_index("s") * NC + lax.axis_index("c")
        b0 = wid * B_PER_W
        pltpu.sync_copy(idx_hbm.at[pl.ds(wid * IDX_PER_W, IDX_PER_W)], idx_v)

        def fire(r, buf, sem):
            base = r // 4 * CHUNK * 4 + (r % 4) * CHUNK
            for p in range(P):
                pltpu.async_copy(
                    emb_hbm.at[idx_v.at[pl.ds(p * 640 + base, CHUNK)]],
                    buf.at[:, pl.ds(p * EP, EP)],
                    sem,
                )

        def out_slice(r):
            return out_hbm.at[
                pl.ds(b0 + (r % 4) * CHUNK, CHUNK), pl.ds(r // 4 * PW, PW)
            ]

        def drain(buf, sem):
            pltpu.make_async_copy(
                out_hbm.at[pl.ds(0, CHUNK), pl.ds(0, PW)], buf, sem
            ).wait()

        fire(0, buf0, g0)
        fire(1, buf1, g1)

        @pl.loop(0, ROUNDS // 2)
        def _(g):
            r0 = 2 * g
            r1 = r0 + 1
            drain(buf0, g0)
            pltpu.async_copy(buf0, out_slice(r0), w0)
            drain(buf1, g1)
            pltpu.async_copy(buf1, out_slice(r1), w1)

            @pl.when(r0 + 2 < ROUNDS)
            def _():
                drain(buf0, w0)
                fire(r0 + 2, buf0, g0)

            @pl.when(r1 + 2 < ROUNDS)
            def _():
                drain(buf1, w1)
                fire(r1 + 2, buf1, g1)

        drain(buf0, w0)
        drain(buf1, w1)

    return k(emb_pad, idx_r)


def _mm_body(x_ref, w_ref, b_ref, o_ref):
    acc = jnp.dot(
        x_ref[...].astype(jnp.bfloat16),
        w_ref[...],
        preferred_element_type=jnp.float32,
    )
    o_ref[...] = jnp.maximum(acc + b_ref[...], 0.0)


def _tc_matmul(g, Wp, b):
    return pl.pallas_call(
        _mm_body,
        grid=(B // BM,),
        in_specs=[
            pl.BlockSpec((BM, K), lambda i: (i, 0)),
            pl.BlockSpec((K, H), lambda i: (0, 0)),
            pl.BlockSpec((1, H), lambda i: (0, 0)),
        ],
        out_specs=pl.BlockSpec((BM, H), lambda i: (i, 0)),
        out_shape=jax.ShapeDtypeStruct((B, H), jnp.float32),
    )(g, Wp, b.reshape(1, H))


def kernel(x, emb, W, b):
    idx_r = (
        x.astype(jnp.int32)
        .reshape(NW, B_PER_W, LP, P)
        .transpose(0, 3, 2, 1)
        .reshape(-1)
    )
    emb_pad = jnp.pad(emb, ((0, 0), (0, EP - E)))
    Wp = jnp.concatenate(
        [
            W.reshape(L, E, H).astype(jnp.bfloat16),
            jnp.zeros((L, EP - E, H), jnp.bfloat16),
        ],
        axis=1,
    ).reshape(K, H)
    g = _sc_gather(emb_pad, idx_r)
    return _tc_matmul(g, Wp, b)
